# baseline (speedup 1.0000x reference)
import functools

import jax
import jax.numpy as jnp
from jax import lax
from jax.experimental import pallas as pl
from jax.experimental.pallas import tpu as pltpu
from jax.experimental.pallas import tpu_sc as plsc

NC = 2
NS = 16
NW = NC * NS
CH = 128


def _sc_aggregate(q, idx_r, ones_h, npad, d, nchunk, with_deg):
  mesh = plsc.VectorSubcoreMesh(core_axis_name="c", subcore_axis_name="s",
                                num_cores=NC, num_subcores=NS)
  rows_per_tile = npad // NS

  out_type = [jax.ShapeDtypeStruct((NC, npad, d), jnp.float32)]
  scratch = [
      pltpu.VMEM((4, 2, CH), jnp.int32),
      pltpu.VMEM((2, CH, d), jnp.float32),
      pltpu.VMEM((16, d), jnp.float32),
      pltpu.VMEM_SHARED((npad, d), jnp.float32),
      pltpu.SemaphoreType.DMA((4,)),
      pltpu.SemaphoreType.DMA((2,)),
  ]
  if with_deg:
    out_type.append(jax.ShapeDtypeStruct((NC, npad), jnp.float32))
    scratch += [
        pltpu.VMEM((CH,), jnp.float32),
        pltpu.VMEM((rows_per_tile,), jnp.float32),
        pltpu.VMEM_SHARED((npad,), jnp.float32),
    ]

  @functools.partial(
      pl.kernel, mesh=mesh, out_type=tuple(out_type),
      scratch_types=tuple(scratch),
      name="agg_deg" if with_deg else "agg")
  def agg_kernel(q_hbm, idx_hbm, ones_hbm, *rest):
    if with_deg:
      (part_hbm, degp_hbm, idx_v, rows_v, zmat_v, acc_s,
       isems, rsems, ones_v, zrow_v, dega_s) = rest
    else:
      (part_hbm, idx_v, rows_v, zmat_v, acc_s, isems, rsems) = rest
    c = lax.axis_index("c")
    s = lax.axis_index("s")
    wid = s * NC + c

    z16 = jnp.zeros((16,), jnp.float32)
    for i in range(16):
      for j in range(d // 16):
        zmat_v[i, pl.ds(j * 16, 16)] = z16
    base = s * rows_per_tile

    def clear_body(k, _):
      pltpu.sync_copy(zmat_v, acc_s.at[pl.ds(base + k * 16, 16)])
      return 0
    lax.fori_loop(0, rows_per_tile // 16, clear_body, 0)

    if with_deg:
      pltpu.sync_copy(ones_hbm, ones_v)
      for j in range(rows_per_tile // 16):
        zrow_v[pl.ds(j * 16, 16)] = z16
      pltpu.sync_copy(zrow_v, dega_s.at[pl.ds(base, rows_per_tile)])

    plsc.subcore_barrier()

    def idx_load(j, slot):
      pltpu.async_copy(idx_hbm.at[wid, j], idx_v.at[slot], isems.at[slot])

    def idx_wait(j, slot):
      pltpu.make_async_copy(idx_hbm.at[wid, j], idx_v.at[slot],
                            isems.at[slot]).wait()

    def gather(j, i4, k2):
      pltpu.async_copy(q_hbm.at[idx_v.at[i4, 0]], rows_v.at[k2],
                       rsems.at[k2])

    def prime_idx(j, _):
      @pl.when(j < nchunk)
      def _():
        idx_load(j, j)
      return 0
    lax.fori_loop(0, 4, prime_idx, 0)

    def prime_rows(j, _):
      @pl.when(j < nchunk)
      def _():
        idx_wait(j, j)
        gather(j, j, j)
      return 0
    lax.fori_loop(0, 2, prime_rows, 0)

    def chunk_body(j, _):
      k2 = lax.rem(j, 2)
      i4 = lax.rem(j, 4)
      pltpu.make_async_copy(q_hbm.at[idx_v.at[i4, 0]], rows_v.at[k2],
                            rsems.at[k2]).wait()
      pltpu.sync_copy(rows_v.at[k2], acc_s.at[idx_v.at[i4, 1]], add=True)
      if with_deg:
        pltpu.sync_copy(ones_v, dega_s.at[idx_v.at[i4, 1]], add=True)

      @pl.when(j + 4 < nchunk)
      def _():
        idx_load(j + 4, i4)

      @pl.when(j + 2 < nchunk)
      def _():
        i4n = lax.rem(j + 2, 4)
        idx_wait(j + 2, i4n)
        gather(j + 2, i4n, k2)
      return 0
    lax.fori_loop(0, nchunk, chunk_body, 0)

    plsc.subcore_barrier()

    pltpu.sync_copy(acc_s.at[pl.ds(base, rows_per_tile)],
                    part_hbm.at[c, pl.ds(base, rows_per_tile)])
    if with_deg:
      pltpu.sync_copy(dega_s.at[pl.ds(base, rows_per_tile)],
                      degp_hbm.at[c, pl.ds(base, rows_per_tile)])

  return agg_kernel(q, idx_r, ones_h)


def _tc_transform(x, Wl, Wr):
  n, _ = x.shape
  dout = Wl.shape[1]

  def body(x_ref, wl_ref, wr_ref, q_ref, r_ref):
    xv = x_ref[...]
    q_ref[...] = jnp.dot(xv, wl_ref[...], preferred_element_type=jnp.float32)
    r_ref[...] = jnp.dot(xv, wr_ref[...], preferred_element_type=jnp.float32)

  return pl.pallas_call(
      body,
      out_shape=(jax.ShapeDtypeStruct((n, dout), jnp.float32),
                 jax.ShapeDtypeStruct((n, dout), jnp.float32)),
  )(x, Wl, Wr)


def _tc_mid(part, deg2, r1, b1):
  n, d = r1.shape

  def body(p_ref, d_ref, r1_ref, b1_ref, h_ref):
    deg = jnp.maximum(d_ref[0] + d_ref[1], 1.0)
    h = (p_ref[0] + p_ref[1]) / deg + b1_ref[...] + r1_ref[...]
    h_ref[...] = jnp.maximum(h, 0.0)

  return pl.pallas_call(
      body,
      out_shape=jax.ShapeDtypeStruct((n, d), jnp.float32),
  )(part, deg2, r1, b1.reshape(1, -1))


def _tc_final(part, deg2, h, b2, Wl2, Wr2):
  n = h.shape[0]
  dout = Wl2.shape[1]

  def body(p_ref, d_ref, h_ref, b2_ref, wl_ref, wr_ref, o_ref):
    deg = jnp.maximum(d_ref[0] + d_ref[1], 1.0)
    mean2 = (p_ref[0] + p_ref[1]) / deg
    o = (jnp.dot(mean2, wl_ref[...], preferred_element_type=jnp.float32)
         + b2_ref[...]
         + jnp.dot(h_ref[...], wr_ref[...],
                   preferred_element_type=jnp.float32))
    m = jnp.max(o, axis=-1, keepdims=True)
    e = jnp.exp(o - m)
    lse = jnp.log(jnp.sum(e, axis=-1, keepdims=True)) + m
    o_ref[...] = o - lse

  return pl.pallas_call(
      body,
      out_shape=jax.ShapeDtypeStruct((n, dout), jnp.float32),
  )(part, deg2, h, b2.reshape(1, -1), Wl2, Wr2)


def kernel(x, edge_index, Wl1, Wr1, b1, Wl2, Wr2, b2):
  n, d_in = x.shape
  e = edge_index.shape[1]
  nchunk = -(-e // (NW * CH))
  e_pad = NW * CH * nchunk
  npad = ((n + NW * 16 - 1) // (NW * 16)) * (NW * 16)
  if e_pad > e and npad == n:
    npad += NW * 16

  xp = jnp.pad(x, ((0, npad - n), (0, 0)))
  src_r = jnp.pad(edge_index[0], (0, e_pad - e)).reshape(NW, nchunk, 1, CH)
  dst_r = jnp.pad(edge_index[1], (0, e_pad - e),
                  constant_values=npad - 1).reshape(NW, nchunk, 1, CH)
  idx_r = jnp.concatenate([src_r, dst_r], axis=2)
  ones_h = jnp.ones((CH,), jnp.float32)

  q1, r1 = _tc_transform(xp, Wl1, Wr1)
  part1, degp = _sc_aggregate(q1, idx_r, ones_h, npad, d_in,
                              nchunk, with_deg=True)
  deg2 = degp.reshape(NC, npad, 1)
  h = _tc_mid(part1, deg2, r1, b1)

  (part2,) = _sc_aggregate(h, idx_r, ones_h, npad, d_in,
                           nchunk, with_deg=False)
  out = _tc_final(part2, deg2, h, b2, Wl2, Wr2)
  return out[:n]


# speedup vs baseline: 1.0493x; 1.0493x over previous
import functools

import jax
import jax.numpy as jnp
from jax import lax
from jax.experimental import pallas as pl
from jax.experimental.pallas import tpu as pltpu
from jax.experimental.pallas import tpu_sc as plsc

NC = 2
NS = 16
NW = NC * NS
CH = 96


def _sc_aggregate(q, idx_r, ones_h, npad, d, nchunk, with_deg):
  mesh = plsc.VectorSubcoreMesh(core_axis_name="c", subcore_axis_name="s",
                                num_cores=NC, num_subcores=NS)
  rows_per_tile = npad // NS

  out_type = [jax.ShapeDtypeStruct((NC, npad, d), jnp.float32)]
  scratch = [
      pltpu.VMEM((8, 2, CH), jnp.int32),
      pltpu.VMEM((3, CH, d), jnp.float32),
      pltpu.VMEM((16, d), jnp.float32),
      pltpu.VMEM_SHARED((npad, d), jnp.float32),
      pltpu.SemaphoreType.DMA((8,)),
      pltpu.SemaphoreType.DMA((3,)),
      pltpu.SemaphoreType.DMA((3,)),
  ]
  if with_deg:
    out_type.append(jax.ShapeDtypeStruct((NC, npad), jnp.float32))
    scratch += [
        pltpu.VMEM((CH,), jnp.float32),
        pltpu.VMEM((rows_per_tile,), jnp.float32),
        pltpu.VMEM_SHARED((npad,), jnp.float32),
    ]

  @functools.partial(
      pl.kernel, mesh=mesh, out_type=tuple(out_type),
      scratch_types=tuple(scratch),
      name="agg_deg" if with_deg else "agg")
  def agg_kernel(q_hbm, idx_hbm, ones_hbm, *rest):
    if with_deg:
      (part_hbm, degp_hbm, idx_v, rows_v, zmat_v, acc_s,
       isems, gsems, ssems, ones_v, zrow_v, dega_s) = rest
    else:
      (part_hbm, idx_v, rows_v, zmat_v, acc_s, isems, gsems, ssems) = rest
    c = lax.axis_index("c")
    s = lax.axis_index("s")
    wid = s * NC + c

    z16 = jnp.zeros((16,), jnp.float32)
    for i in range(16):
      for j in range(d // 16):
        zmat_v[i, pl.ds(j * 16, 16)] = z16
    base = s * rows_per_tile

    def clear_body(k, _):
      pltpu.sync_copy(zmat_v, acc_s.at[pl.ds(base + k * 16, 16)])
      return 0
    lax.fori_loop(0, rows_per_tile // 16, clear_body, 0)

    if with_deg:
      pltpu.sync_copy(ones_hbm, ones_v)
      for j in range(rows_per_tile // 16):
        zrow_v[pl.ds(j * 16, 16)] = z16
      pltpu.sync_copy(zrow_v, dega_s.at[pl.ds(base, rows_per_tile)])

    plsc.subcore_barrier()

    def idx_load(j, slot):
      pltpu.async_copy(idx_hbm.at[wid, j], idx_v.at[slot], isems.at[slot])

    def idx_wait(j, slot):
      pltpu.make_async_copy(idx_hbm.at[wid, j], idx_v.at[slot],
                            isems.at[slot]).wait()

    def gather_start(j, k3):
      pltpu.async_copy(q_hbm.at[idx_v.at[lax.rem(j, 8), 0]],
                       rows_v.at[k3], gsems.at[k3])

    def gather_wait(j, k3):
      pltpu.make_async_copy(q_hbm.at[idx_v.at[lax.rem(j, 8), 0]],
                            rows_v.at[k3], gsems.at[k3]).wait()

    def scat_start(j, k3):
      pltpu.async_copy(rows_v.at[k3], acc_s.at[idx_v.at[lax.rem(j, 8), 1]],
                       ssems.at[k3], add=True)

    def scat_wait(j, k3):
      pltpu.make_async_copy(rows_v.at[k3],
                            acc_s.at[idx_v.at[lax.rem(j, 8), 1]],
                            ssems.at[k3]).wait()

    def prime_idx(j, _):
      @pl.when(j < nchunk)
      def _():
        idx_load(j, j)
      return 0
    lax.fori_loop(0, 8, prime_idx, 0)

    def prime_rows(j, _):
      @pl.when(j < nchunk)
      def _():
        idx_wait(j, j)
        gather_start(j, j)
      return 0
    lax.fori_loop(0, 2, prime_rows, 0)

    def chunk_body(j, _):
      k3 = lax.rem(j, 3)
      gather_wait(j, k3)
      scat_start(j, k3)
      if with_deg:
        pltpu.sync_copy(ones_v, dega_s.at[idx_v.at[lax.rem(j, 8), 1]],
                        add=True)

      @pl.when(j >= 1)
      def _():
        scat_wait(j - 1, lax.rem(j - 1, 3))

        @pl.when(j + 7 < nchunk)
        def _():
          idx_load(j + 7, lax.rem(j + 7, 8))

      @pl.when(j + 2 < nchunk)
      def _():
        idx_wait(j + 2, lax.rem(j + 2, 8))
        gather_start(j + 2, lax.rem(j + 2, 3))
      return 0
    lax.fori_loop(0, nchunk, chunk_body, 0)

    scat_wait(nchunk - 1, lax.rem(nchunk - 1, 3))

    plsc.subcore_barrier()

    pltpu.sync_copy(acc_s.at[pl.ds(base, rows_per_tile)],
                    part_hbm.at[c, pl.ds(base, rows_per_tile)])
    if with_deg:
      pltpu.sync_copy(dega_s.at[pl.ds(base, rows_per_tile)],
                      degp_hbm.at[c, pl.ds(base, rows_per_tile)])

  return agg_kernel(q, idx_r, ones_h)


def _tc_transform(x, Wl, Wr):
  n, _ = x.shape
  dout = Wl.shape[1]

  def body(x_ref, wl_ref, wr_ref, q_ref, r_ref):
    xv = x_ref[...]
    q_ref[...] = jnp.dot(xv, wl_ref[...], preferred_element_type=jnp.float32)
    r_ref[...] = jnp.dot(xv, wr_ref[...], preferred_element_type=jnp.float32)

  return pl.pallas_call(
      body,
      out_shape=(jax.ShapeDtypeStruct((n, dout), jnp.float32),
                 jax.ShapeDtypeStruct((n, dout), jnp.float32)),
  )(x, Wl, Wr)


def _tc_mid(part, deg2, r1, b1):
  n, d = r1.shape

  def body(p_ref, d_ref, r1_ref, b1_ref, h_ref):
    deg = jnp.maximum(d_ref[0] + d_ref[1], 1.0)
    h = (p_ref[0] + p_ref[1]) / deg + b1_ref[...] + r1_ref[...]
    h_ref[...] = jnp.maximum(h, 0.0)

  return pl.pallas_call(
      body,
      out_shape=jax.ShapeDtypeStruct((n, d), jnp.float32),
  )(part, deg2, r1, b1.reshape(1, -1))


def _tc_final(part, deg2, h, b2, Wl2, Wr2):
  n = h.shape[0]
  dout = Wl2.shape[1]

  def body(p_ref, d_ref, h_ref, b2_ref, wl_ref, wr_ref, o_ref):
    deg = jnp.maximum(d_ref[0] + d_ref[1], 1.0)
    mean2 = (p_ref[0] + p_ref[1]) / deg
    o = (jnp.dot(mean2, wl_ref[...], preferred_element_type=jnp.float32)
         + b2_ref[...]
         + jnp.dot(h_ref[...], wr_ref[...],
                   preferred_element_type=jnp.float32))
    m = jnp.max(o, axis=-1, keepdims=True)
    e = jnp.exp(o - m)
    lse = jnp.log(jnp.sum(e, axis=-1, keepdims=True)) + m
    o_ref[...] = o - lse

  return pl.pallas_call(
      body,
      out_shape=jax.ShapeDtypeStruct((n, dout), jnp.float32),
  )(part, deg2, h, b2.reshape(1, -1), Wl2, Wr2)


def kernel(x, edge_index, Wl1, Wr1, b1, Wl2, Wr2, b2):
  n, d_in = x.shape
  e = edge_index.shape[1]
  nchunk = -(-e // (NW * CH))
  e_pad = NW * CH * nchunk
  npad = ((n + NW * 16 - 1) // (NW * 16)) * (NW * 16)
  if e_pad > e and npad == n:
    npad += NW * 16

  xp = jnp.pad(x, ((0, npad - n), (0, 0)))
  src_r = jnp.pad(edge_index[0], (0, e_pad - e)).reshape(NW, nchunk, 1, CH)
  dst_r = jnp.pad(edge_index[1], (0, e_pad - e),
                  constant_values=npad - 1).reshape(NW, nchunk, 1, CH)
  idx_r = jnp.concatenate([src_r, dst_r], axis=2)
  ones_h = jnp.ones((CH,), jnp.float32)

  q1, r1 = _tc_transform(xp, Wl1, Wr1)
  part1, degp = _sc_aggregate(q1, idx_r, ones_h, npad, d_in,
                              nchunk, with_deg=True)
  deg2 = degp.reshape(NC, npad, 1)
  h = _tc_mid(part1, deg2, r1, b1)

  (part2,) = _sc_aggregate(h, idx_r, ones_h, npad, d_in,
                           nchunk, with_deg=False)
  out = _tc_final(part2, deg2, h, b2, Wl2, Wr2)
  return out[:n]


# speedup vs baseline: 1.4297x; 1.3625x over previous
import functools

import jax
import jax.numpy as jnp
from jax import lax
from jax.experimental import pallas as pl
from jax.experimental.pallas import tpu as pltpu
from jax.experimental.pallas import tpu_sc as plsc

NC = 2
NS = 16
NW = NC * NS
CH = 125


def _sc_aggregate(q, src_r, dst_r, ones_h, npad, d, nchunk, with_deg):
  mesh = plsc.VectorSubcoreMesh(core_axis_name="c", subcore_axis_name="s",
                                num_cores=NC, num_subcores=NS)
  rows_per_tile = npad // NS

  out_type = [jax.ShapeDtypeStruct((NC, npad, d), jnp.float32)]
  scratch = [
      pltpu.VMEM((nchunk, CH), jnp.int32),
      pltpu.VMEM((nchunk, CH), jnp.int32),
      pltpu.VMEM((CH, d), jnp.float32),
      pltpu.VMEM((16, d), jnp.float32),
      pltpu.VMEM_SHARED((npad, d), jnp.float32),
      pltpu.SemaphoreType.DMA,
  ]
  if with_deg:
    out_type.append(jax.ShapeDtypeStruct((NC, npad), jnp.float32))
    scratch += [
        pltpu.VMEM((CH,), jnp.float32),
        pltpu.VMEM((rows_per_tile,), jnp.float32),
        pltpu.VMEM_SHARED((npad,), jnp.float32),
    ]

  @functools.partial(
      pl.kernel, mesh=mesh, out_type=tuple(out_type),
      scratch_types=tuple(scratch),
      compiler_params=pltpu.CompilerParams(use_tc_tiling_on_sc=False),
      name="agg_deg" if with_deg else "agg")
  def agg_kernel(q_hbm, src_hbm, dst_hbm, ones_hbm, *rest):
    if with_deg:
      (part_hbm, degp_hbm, src_v, dst_v, rows_v, zmat_v, acc_s, sem,
       ones_v, zrow_v, dega_s) = rest
    else:
      (part_hbm, src_v, dst_v, rows_v, zmat_v, acc_s, sem) = rest
    c = lax.axis_index("c")
    s = lax.axis_index("s")
    wid = s * NC + c

    pltpu.sync_copy(src_hbm.at[wid], src_v)
    pltpu.sync_copy(dst_hbm.at[wid], dst_v)

    z16 = jnp.zeros((16,), jnp.float32)
    for i in range(16):
      for j in range(d // 16):
        zmat_v[i, pl.ds(j * 16, 16)] = z16
    base = s * rows_per_tile

    def clear_body(k, _):
      pltpu.sync_copy(zmat_v, acc_s.at[pl.ds(base + k * 16, 16)])
      return 0
    lax.fori_loop(0, rows_per_tile // 16, clear_body, 0)

    if with_deg:
      pltpu.sync_copy(ones_hbm, ones_v)
      for j in range(rows_per_tile // 16):
        zrow_v[pl.ds(j * 16, 16)] = z16
      pltpu.sync_copy(zrow_v, dega_s.at[pl.ds(base, rows_per_tile)])

    plsc.subcore_barrier()

    def chunk_body(j, _):
      pltpu.async_copy(q_hbm.at[src_v.at[j]], rows_v, sem).wait()
      pltpu.sync_copy(rows_v, acc_s.at[dst_v.at[j]], add=True)
      if with_deg:
        pltpu.sync_copy(ones_v, dega_s.at[dst_v.at[j]], add=True)
      return 0
    lax.fori_loop(0, nchunk, chunk_body, 0)

    plsc.subcore_barrier()

    pltpu.sync_copy(acc_s.at[pl.ds(base, rows_per_tile)],
                    part_hbm.at[c, pl.ds(base, rows_per_tile)])
    if with_deg:
      pltpu.sync_copy(dega_s.at[pl.ds(base, rows_per_tile)],
                      degp_hbm.at[c, pl.ds(base, rows_per_tile)])

  return agg_kernel(q, src_r, dst_r, ones_h)


def _tc_transform(x, Wl, Wr):
  n, _ = x.shape
  dout = Wl.shape[1]

  def body(x_ref, wl_ref, wr_ref, q_ref, r_ref):
    xv = x_ref[...]
    q_ref[...] = jnp.dot(xv, wl_ref[...], preferred_element_type=jnp.float32)
    r_ref[...] = jnp.dot(xv, wr_ref[...], preferred_element_type=jnp.float32)

  return pl.pallas_call(
      body,
      out_shape=(jax.ShapeDtypeStruct((n, dout), jnp.float32),
                 jax.ShapeDtypeStruct((n, dout), jnp.float32)),
  )(x, Wl, Wr)


def _tc_mid(part, deg2, r1, b1, Wl2, Wr2):
  n = r1.shape[0]
  dout = Wl2.shape[1]

  def body(p_ref, d_ref, r1_ref, b1_ref, wl_ref, wr_ref, q2_ref, r2_ref):
    deg = jnp.maximum(d_ref[0] + d_ref[1], 1.0)
    h = (p_ref[0] + p_ref[1]) / deg + b1_ref[...] + r1_ref[...]
    h = jnp.maximum(h, 0.0)
    q2_ref[...] = jnp.dot(h, wl_ref[...], preferred_element_type=jnp.float32)
    r2_ref[...] = jnp.dot(h, wr_ref[...], preferred_element_type=jnp.float32)

  return pl.pallas_call(
      body,
      out_shape=(jax.ShapeDtypeStruct((n, dout), jnp.float32),
                 jax.ShapeDtypeStruct((n, dout), jnp.float32)),
  )(part, deg2, r1, b1.reshape(1, -1), Wl2, Wr2)


def _tc_final(part, deg2, r2, b2):
  n, dout = r2.shape

  def body(p_ref, d_ref, r2_ref, b2_ref, o_ref):
    deg = jnp.maximum(d_ref[0] + d_ref[1], 1.0)
    o = (p_ref[0] + p_ref[1]) / deg + b2_ref[...] + r2_ref[...]
    m = jnp.max(o, axis=-1, keepdims=True)
    e = jnp.exp(o - m)
    lse = jnp.log(jnp.sum(e, axis=-1, keepdims=True)) + m
    o_ref[...] = o - lse

  return pl.pallas_call(
      body,
      out_shape=jax.ShapeDtypeStruct((n, dout), jnp.float32),
  )(part, deg2, r2, b2.reshape(1, -1))


def kernel(x, edge_index, Wl1, Wr1, b1, Wl2, Wr2, b2):
  n, d_in = x.shape
  e = edge_index.shape[1]
  nchunk = -(-e // (NW * CH))
  e_pad = NW * CH * nchunk
  npad = ((n + NW * 16 - 1) // (NW * 16)) * (NW * 16)
  if e_pad > e and npad == n:
    npad += NW * 16

  xp = jnp.pad(x, ((0, npad - n), (0, 0)))
  src_r = jnp.pad(edge_index[0], (0, e_pad - e)).reshape(NW, nchunk, CH)
  dst_r = jnp.pad(edge_index[1], (0, e_pad - e),
                  constant_values=npad - 1).reshape(NW, nchunk, CH)
  ones_h = jnp.ones((CH,), jnp.float32)

  q1, r1 = _tc_transform(xp, Wl1, Wr1)
  part1, degp = _sc_aggregate(q1, src_r, dst_r, ones_h, npad, d_in,
                              nchunk, with_deg=True)
  deg2 = degp.reshape(NC, npad, 1)
  q2, r2 = _tc_mid(part1, deg2, r1, b1, Wl2, Wr2)

  (part2,) = _sc_aggregate(q2, src_r, dst_r, ones_h, npad, Wl2.shape[1],
                           nchunk, with_deg=False)
  out = _tc_final(part2, deg2, r2, b2)
  return out[:n]


# speedup vs baseline: 1.4956x; 1.0461x over previous
import functools

import jax
import jax.numpy as jnp
from jax import lax
from jax.experimental import pallas as pl
from jax.experimental.pallas import tpu as pltpu
from jax.experimental.pallas import tpu_sc as plsc

NC = 2
NS = 16
NW = NC * NS
CH1 = 96
CH2 = 125


def _sc_aggregate(q, src_r, dst_r, ones_h, npad, d, nchunk, ch, with_deg):
  mesh = plsc.VectorSubcoreMesh(core_axis_name="c", subcore_axis_name="s",
                                num_cores=NC, num_subcores=NS)
  rows_per_tile = npad // NS

  out_type = [jax.ShapeDtypeStruct((NC, npad, d), jnp.float32)]
  scratch = [
      pltpu.VMEM((nchunk, ch), jnp.int32),
      pltpu.VMEM((nchunk, ch), jnp.int32),
      pltpu.VMEM((2, ch, d), jnp.float32),
      pltpu.VMEM((16, d), jnp.float32),
      pltpu.VMEM_SHARED((npad, d), jnp.float32),
      pltpu.SemaphoreType.DMA((2,)),
  ]
  if with_deg:
    out_type.append(jax.ShapeDtypeStruct((NC, npad), jnp.float32))
    scratch += [
        pltpu.VMEM((ch,), jnp.float32),
        pltpu.VMEM((rows_per_tile,), jnp.float32),
        pltpu.VMEM_SHARED((npad,), jnp.float32),
        pltpu.SemaphoreType.DMA,
    ]

  @functools.partial(
      pl.kernel, mesh=mesh, out_type=tuple(out_type),
      scratch_types=tuple(scratch),
      compiler_params=pltpu.CompilerParams(use_tc_tiling_on_sc=False),
      name="agg_deg" if with_deg else "agg")
  def agg_kernel(q_hbm, src_hbm, dst_hbm, ones_hbm, *rest):
    if with_deg:
      (part_hbm, degp_hbm, src_v, dst_v, rows_v, zmat_v, acc_s, gsems,
       ones_v, zrow_v, dega_s, dsem) = rest
    else:
      (part_hbm, src_v, dst_v, rows_v, zmat_v, acc_s, gsems) = rest
    c = lax.axis_index("c")
    s = lax.axis_index("s")
    wid = s * NC + c

    pltpu.sync_copy(src_hbm.at[wid], src_v)
    pltpu.sync_copy(dst_hbm.at[wid], dst_v)

    z16 = jnp.zeros((16,), jnp.float32)
    for i in range(16):
      for j in range(d // 16):
        zmat_v[i, pl.ds(j * 16, 16)] = z16
    base = s * rows_per_tile

    def clear_body(k, _):
      pltpu.sync_copy(zmat_v, acc_s.at[pl.ds(base + k * 16, 16)])
      return 0
    lax.fori_loop(0, rows_per_tile // 16, clear_body, 0)

    if with_deg:
      pltpu.sync_copy(ones_hbm, ones_v)
      for j in range(rows_per_tile // 16):
        zrow_v[pl.ds(j * 16, 16)] = z16
      pltpu.sync_copy(zrow_v, dega_s.at[pl.ds(base, rows_per_tile)])

    plsc.subcore_barrier()

    pltpu.async_copy(q_hbm.at[src_v.at[0]], rows_v.at[0], gsems.at[0])

    def chunk_body(j, _):
      k = lax.rem(j, 2)

      @pl.when(j + 1 < nchunk)
      def _():
        pltpu.async_copy(q_hbm.at[src_v.at[j + 1]], rows_v.at[1 - k],
                         gsems.at[1 - k])

      pltpu.make_async_copy(q_hbm.at[src_v.at[j]], rows_v.at[k],
                            gsems.at[k]).wait()
      pltpu.sync_copy(rows_v.at[k], acc_s.at[dst_v.at[j]], add=True)
      if with_deg:
        pltpu.async_copy(ones_v, dega_s.at[dst_v.at[j]], dsem, add=True)
      return 0
    lax.fori_loop(0, nchunk, chunk_body, 0)

    if with_deg:
      pltpu.make_async_copy(src_hbm.at[wid], dst_v, dsem).wait()

    plsc.subcore_barrier()

    pltpu.sync_copy(acc_s.at[pl.ds(base, rows_per_tile)],
                    part_hbm.at[c, pl.ds(base, rows_per_tile)])
    if with_deg:
      pltpu.sync_copy(dega_s.at[pl.ds(base, rows_per_tile)],
                      degp_hbm.at[c, pl.ds(base, rows_per_tile)])

  return agg_kernel(q, src_r, dst_r, ones_h)


def _tc_transform(x, Wl, Wr):
  n, _ = x.shape
  dout = Wl.shape[1]

  def body(x_ref, wl_ref, wr_ref, q_ref, r_ref):
    xv = x_ref[...]
    q_ref[...] = jnp.dot(xv, wl_ref[...], preferred_element_type=jnp.float32)
    r_ref[...] = jnp.dot(xv, wr_ref[...], preferred_element_type=jnp.float32)

  return pl.pallas_call(
      body,
      out_shape=(jax.ShapeDtypeStruct((n, dout), jnp.float32),
                 jax.ShapeDtypeStruct((n, dout), jnp.float32)),
  )(x, Wl, Wr)


def _tc_mid(part, deg2, r1, b1, Wl2, Wr2):
  n = r1.shape[0]
  dout = Wl2.shape[1]

  def body(p_ref, d_ref, r1_ref, b1_ref, wl_ref, wr_ref, q2_ref, r2_ref):
    deg = jnp.maximum(d_ref[0] + d_ref[1], 1.0)
    h = (p_ref[0] + p_ref[1]) / deg + b1_ref[...] + r1_ref[...]
    h = jnp.maximum(h, 0.0)
    q2_ref[...] = jnp.dot(h, wl_ref[...], preferred_element_type=jnp.float32)
    r2_ref[...] = jnp.dot(h, wr_ref[...], preferred_element_type=jnp.float32)

  return pl.pallas_call(
      body,
      out_shape=(jax.ShapeDtypeStruct((n, dout), jnp.float32),
                 jax.ShapeDtypeStruct((n, dout), jnp.float32)),
  )(part, deg2, r1, b1.reshape(1, -1), Wl2, Wr2)


def _tc_final(part, deg2, r2, b2):
  n, dout = r2.shape

  def body(p_ref, d_ref, r2_ref, b2_ref, o_ref):
    deg = jnp.maximum(d_ref[0] + d_ref[1], 1.0)
    o = (p_ref[0] + p_ref[1]) / deg + b2_ref[...] + r2_ref[...]
    m = jnp.max(o, axis=-1, keepdims=True)
    e = jnp.exp(o - m)
    lse = jnp.log(jnp.sum(e, axis=-1, keepdims=True)) + m
    o_ref[...] = o - lse

  return pl.pallas_call(
      body,
      out_shape=jax.ShapeDtypeStruct((n, dout), jnp.float32),
  )(part, deg2, r2, b2.reshape(1, -1))


def _edge_chunks(edge_index, ch, npad):
  e = edge_index.shape[1]
  nchunk = -(-e // (NW * ch))
  e_pad = NW * ch * nchunk
  src_r = jnp.pad(edge_index[0], (0, e_pad - e)).reshape(NW, nchunk, ch)
  dst_r = jnp.pad(edge_index[1], (0, e_pad - e),
                  constant_values=npad - 1).reshape(NW, nchunk, ch)
  return src_r, dst_r, nchunk


def kernel(x, edge_index, Wl1, Wr1, b1, Wl2, Wr2, b2):
  n, d_in = x.shape
  npad = ((n + NW * 16 - 1) // (NW * 16)) * (NW * 16)
  if npad == n:
    npad += NW * 16

  xp = jnp.pad(x, ((0, npad - n), (0, 0)))
  src1, dst1, nch1 = _edge_chunks(edge_index, CH1, npad)
  src2, dst2, nch2 = _edge_chunks(edge_index, CH2, npad)

  q1, r1 = _tc_transform(xp, Wl1, Wr1)
  part1, degp = _sc_aggregate(q1, src1, dst1, jnp.ones((CH1,), jnp.float32),
                              npad, d_in, nch1, CH1, with_deg=True)
  deg2 = degp.reshape(NC, npad, 1)
  q2, r2 = _tc_mid(part1, deg2, r1, b1, Wl2, Wr2)

  (part2,) = _sc_aggregate(q2, src2, dst2, jnp.ones((CH2,), jnp.float32),
                           npad, Wl2.shape[1], nch2, CH2, with_deg=False)
  out = _tc_final(part2, deg2, r2, b2)
  return out[:n]


# speedup vs baseline: 1.6235x; 1.0855x over previous
import functools

import jax
import jax.numpy as jnp
from jax import lax
from jax.experimental import pallas as pl
from jax.experimental.pallas import tpu as pltpu
from jax.experimental.pallas import tpu_sc as plsc

NC = 2
NS = 16
NW = NC * NS
CH = 125


def _sc_aggregate(q, src_r, dst_r, ones_h, npad, d, nchunk, ch, slots,
                  with_deg):
  mesh = plsc.VectorSubcoreMesh(core_axis_name="c", subcore_axis_name="s",
                                num_cores=NC, num_subcores=NS)
  rows_per_tile = npad // NS

  out_type = [jax.ShapeDtypeStruct((NC, npad, d), jnp.float32)]
  scratch = [
      pltpu.VMEM((nchunk, ch), jnp.int32),
      pltpu.VMEM((nchunk, ch), jnp.int32),
      pltpu.VMEM((slots, ch, d), jnp.float32),
      pltpu.VMEM((16, d), jnp.float32),
      pltpu.VMEM_SHARED((npad, d), jnp.float32),
      pltpu.SemaphoreType.DMA((2,)),
  ]
  if with_deg:
    out_type.append(jax.ShapeDtypeStruct((NC, npad), jnp.float32))
    scratch += [
        pltpu.VMEM((ch,), jnp.float32),
        pltpu.VMEM((rows_per_tile,), jnp.float32),
        pltpu.VMEM_SHARED((npad,), jnp.float32),
        pltpu.SemaphoreType.DMA,
    ]

  @functools.partial(
      pl.kernel, mesh=mesh, out_type=tuple(out_type),
      scratch_types=tuple(scratch),
      compiler_params=pltpu.CompilerParams(use_tc_tiling_on_sc=False),
      name="agg_deg" if with_deg else "agg")
  def agg_kernel(q_hbm, src_hbm, dst_hbm, ones_hbm, *rest):
    if with_deg:
      (part_hbm, degp_hbm, src_v, dst_v, rows_v, zmat_v, acc_s, gsems,
       ones_v, zrow_v, dega_s, dsem) = rest
    else:
      (part_hbm, src_v, dst_v, rows_v, zmat_v, acc_s, gsems) = rest
    c = lax.axis_index("c")
    s = lax.axis_index("s")
    wid = s * NC + c

    pltpu.sync_copy(src_hbm.at[wid], src_v)
    pltpu.sync_copy(dst_hbm.at[wid], dst_v)

    z16 = jnp.zeros((16,), jnp.float32)
    for i in range(16):
      for j in range(d // 16):
        zmat_v[i, pl.ds(j * 16, 16)] = z16
    base = s * rows_per_tile

    def clear_body(k, _):
      pltpu.sync_copy(zmat_v, acc_s.at[pl.ds(base + k * 16, 16)])
      return 0
    lax.fori_loop(0, rows_per_tile // 16, clear_body, 0)

    if with_deg:
      pltpu.sync_copy(ones_hbm, ones_v)
      for j in range(rows_per_tile // 16):
        zrow_v[pl.ds(j * 16, 16)] = z16
      pltpu.sync_copy(zrow_v, dega_s.at[pl.ds(base, rows_per_tile)])

    plsc.subcore_barrier()

    if slots == 2:
      pltpu.async_copy(q_hbm.at[src_v.at[0]], rows_v.at[0], gsems.at[0])

      def chunk_body(j, _):
        k = lax.rem(j, 2)

        @pl.when(j + 1 < nchunk)
        def _():
          pltpu.async_copy(q_hbm.at[src_v.at[j + 1]], rows_v.at[1 - k],
                           gsems.at[1 - k])

        pltpu.make_async_copy(q_hbm.at[src_v.at[j]], rows_v.at[k],
                              gsems.at[k]).wait()
        pltpu.sync_copy(rows_v.at[k], acc_s.at[dst_v.at[j]], add=True)
        if with_deg:
          pltpu.async_copy(ones_v, dega_s.at[dst_v.at[j]], dsem, add=True)
        return 0
    else:
      def chunk_body(j, _):
        pltpu.async_copy(q_hbm.at[src_v.at[j]], rows_v.at[0],
                         gsems.at[0]).wait()
        pltpu.sync_copy(rows_v.at[0], acc_s.at[dst_v.at[j]], add=True)
        if with_deg:
          pltpu.sync_copy(ones_v, dega_s.at[dst_v.at[j]], add=True)
        return 0
    lax.fori_loop(0, nchunk, chunk_body, 0)

    if with_deg and slots == 2:
      pltpu.make_async_copy(src_hbm.at[wid], dst_v, dsem).wait()

    plsc.subcore_barrier()

    pltpu.sync_copy(acc_s.at[pl.ds(base, rows_per_tile)],
                    part_hbm.at[c, pl.ds(base, rows_per_tile)])
    if with_deg:
      pltpu.sync_copy(dega_s.at[pl.ds(base, rows_per_tile)],
                      degp_hbm.at[c, pl.ds(base, rows_per_tile)])

  return agg_kernel(q, src_r, dst_r, ones_h)


def _tc_transform(x, Wl, Wr):
  n, _ = x.shape
  dout = Wl.shape[1]

  def body(x_ref, wl_ref, wr_ref, q_ref, r_ref):
    xv = x_ref[...]
    q_ref[...] = jnp.dot(xv, wl_ref[...], preferred_element_type=jnp.float32)
    r_ref[...] = jnp.dot(xv, wr_ref[...], preferred_element_type=jnp.float32)

  return pl.pallas_call(
      body,
      out_shape=(jax.ShapeDtypeStruct((n, dout), jnp.float32),
                 jax.ShapeDtypeStruct((n, dout), jnp.float32)),
  )(x, Wl, Wr)


def _tc_mid(part, deg2, r1, b1, Wl2, Wr2):
  n = r1.shape[0]
  dout = Wl2.shape[1]

  def body(p_ref, d_ref, r1_ref, b1_ref, wl_ref, wr_ref, q2_ref, r2_ref):
    deg = jnp.maximum(d_ref[0] + d_ref[1], 1.0)
    h = (p_ref[0] + p_ref[1]) / deg + b1_ref[...] + r1_ref[...]
    h = jnp.maximum(h, 0.0)
    q2_ref[...] = jnp.dot(h, wl_ref[...], preferred_element_type=jnp.float32)
    r2_ref[...] = jnp.dot(h, wr_ref[...], preferred_element_type=jnp.float32)

  return pl.pallas_call(
      body,
      out_shape=(jax.ShapeDtypeStruct((n, dout), jnp.float32),
                 jax.ShapeDtypeStruct((n, dout), jnp.float32)),
  )(part, deg2, r1, b1.reshape(1, -1), Wl2, Wr2)


def _tc_final(part, deg2, r2, b2):
  n, dout = r2.shape

  def body(p_ref, d_ref, r2_ref, b2_ref, o_ref):
    deg = jnp.maximum(d_ref[0] + d_ref[1], 1.0)
    o = (p_ref[0] + p_ref[1]) / deg + b2_ref[...] + r2_ref[...]
    m = jnp.max(o, axis=-1, keepdims=True)
    e = jnp.exp(o - m)
    lse = jnp.log(jnp.sum(e, axis=-1, keepdims=True)) + m
    o_ref[...] = o - lse

  return pl.pallas_call(
      body,
      out_shape=jax.ShapeDtypeStruct((n, dout), jnp.float32),
  )(part, deg2, r2, b2.reshape(1, -1))


def _edge_chunks(edge_index, ch, npad):
  e = edge_index.shape[1]
  nchunk = -(-e // (NW * ch))
  e_pad = NW * ch * nchunk
  src_r = jnp.pad(edge_index[0], (0, e_pad - e)).reshape(NW, nchunk, ch)
  dst_r = jnp.pad(edge_index[1], (0, e_pad - e),
                  constant_values=npad - 1).reshape(NW, nchunk, ch)
  return src_r, dst_r, nchunk


def kernel(x, edge_index, Wl1, Wr1, b1, Wl2, Wr2, b2):
  n, d_in = x.shape
  npad = ((n + NW * 16 - 1) // (NW * 16)) * (NW * 16)
  if npad == n:
    npad += NW * 16

  xp = jnp.pad(x, ((0, npad - n), (0, 0)))
  src_r, dst_r, nchunk = _edge_chunks(edge_index, CH, npad)
  ones_h = jnp.ones((CH,), jnp.float32)

  q1, r1 = _tc_transform(xp, Wl1, Wr1)
  part1, degp = _sc_aggregate(q1, src_r, dst_r, ones_h,
                              npad, d_in, nchunk, CH, 1, with_deg=True)
  deg2 = degp.reshape(NC, npad, 1)
  q2, r2 = _tc_mid(part1, deg2, r1, b1, Wl2, Wr2)

  (part2,) = _sc_aggregate(q2, src_r, dst_r, ones_h,
                           npad, Wl2.shape[1], nchunk, CH, 2,
                           with_deg=False)
  return _tc_final(part2, deg2, r2, b2)[:n]


# speedup vs baseline: 1.6425x; 1.0117x over previous
import functools

import jax
import jax.numpy as jnp
from jax import lax
from jax.experimental import pallas as pl
from jax.experimental.pallas import tpu as pltpu
from jax.experimental.pallas import tpu_sc as plsc

NC = 2
NS = 16
NW = NC * NS
CH = 125


def _sc_aggregate(q, src_r, dst_r, ones_h, npad, d, nchunk, ch, slots,
                  with_deg):
  mesh = plsc.VectorSubcoreMesh(core_axis_name="c", subcore_axis_name="s",
                                num_cores=NC, num_subcores=NS)
  rows_per_tile = npad // NS

  out_type = [jax.ShapeDtypeStruct((NC, npad, d), jnp.float32)]
  scratch = [
      pltpu.VMEM((nchunk, ch), jnp.int32),
      pltpu.VMEM((nchunk, ch), jnp.int32),
      pltpu.VMEM((slots, ch, d), jnp.float32),
      pltpu.VMEM((16, d), jnp.float32),
      pltpu.VMEM_SHARED((npad, d), jnp.float32),
      pltpu.SemaphoreType.DMA((2,)),
  ]
  if with_deg:
    out_type.append(jax.ShapeDtypeStruct((NC, npad), jnp.float32))
    scratch += [
        pltpu.VMEM((ch,), jnp.float32),
        pltpu.VMEM((rows_per_tile,), jnp.float32),
        pltpu.VMEM_SHARED((npad,), jnp.float32),
        pltpu.SemaphoreType.DMA,
    ]

  @functools.partial(
      pl.kernel, mesh=mesh, out_type=tuple(out_type),
      scratch_types=tuple(scratch),
      compiler_params=pltpu.CompilerParams(use_tc_tiling_on_sc=False),
      name="agg_deg" if with_deg else "agg")
  def agg_kernel(q_hbm, src_hbm, dst_hbm, ones_hbm, *rest):
    if with_deg:
      (part_hbm, degp_hbm, src_v, dst_v, rows_v, zmat_v, acc_s, gsems,
       ones_v, zrow_v, dega_s, dsem) = rest
    else:
      (part_hbm, src_v, dst_v, rows_v, zmat_v, acc_s, gsems) = rest
    c = lax.axis_index("c")
    s = lax.axis_index("s")
    wid = s * NC + c

    pltpu.sync_copy(src_hbm.at[wid], src_v)
    pltpu.sync_copy(dst_hbm.at[wid], dst_v)

    z16 = jnp.zeros((16,), jnp.float32)
    for i in range(16):
      for j in range(d // 16):
        zmat_v[i, pl.ds(j * 16, 16)] = z16
    base = s * rows_per_tile

    def clear_body(k, _):
      pltpu.sync_copy(zmat_v, acc_s.at[pl.ds(base + k * 16, 16)])
      return 0
    lax.fori_loop(0, rows_per_tile // 16, clear_body, 0)

    if with_deg:
      pltpu.sync_copy(ones_hbm, ones_v)
      for j in range(rows_per_tile // 16):
        zrow_v[pl.ds(j * 16, 16)] = z16
      pltpu.sync_copy(zrow_v, dega_s.at[pl.ds(base, rows_per_tile)])

    plsc.subcore_barrier()

    if slots == 2:
      pltpu.async_copy(q_hbm.at[src_v.at[0]], rows_v.at[0], gsems.at[0])

      def chunk_body(j, _):
        k = lax.rem(j, 2)

        @pl.when(j + 1 < nchunk)
        def _():
          pltpu.async_copy(q_hbm.at[src_v.at[j + 1]], rows_v.at[1 - k],
                           gsems.at[1 - k])

        pltpu.make_async_copy(q_hbm.at[src_v.at[j]], rows_v.at[k],
                              gsems.at[k]).wait()
        pltpu.sync_copy(rows_v.at[k], acc_s.at[dst_v.at[j]], add=True)
        if with_deg:
          pltpu.async_copy(ones_v, dega_s.at[dst_v.at[j]], dsem, add=True)
        return 0
    else:
      def chunk_body(j, _):
        pltpu.async_copy(q_hbm.at[src_v.at[j]], rows_v.at[0],
                         gsems.at[0]).wait()
        pltpu.sync_copy(rows_v.at[0], acc_s.at[dst_v.at[j]], add=True)
        if with_deg:
          pltpu.sync_copy(ones_v, dega_s.at[dst_v.at[j]], add=True)
        return 0
    lax.fori_loop(0, nchunk, chunk_body, 0)

    if with_deg and slots == 2:
      pltpu.make_async_copy(src_hbm.at[wid], dst_v, dsem).wait()

    plsc.subcore_barrier()

    pltpu.sync_copy(acc_s.at[pl.ds(base, rows_per_tile)],
                    part_hbm.at[c, pl.ds(base, rows_per_tile)])
    if with_deg:
      pltpu.sync_copy(dega_s.at[pl.ds(base, rows_per_tile)],
                      degp_hbm.at[c, pl.ds(base, rows_per_tile)])

  return agg_kernel(q, src_r, dst_r, ones_h)


def _tc_transform(x, Wl, Wr):
  n, _ = x.shape
  dout = Wl.shape[1]

  def body(x_ref, wl_ref, wr_ref, q_ref, r_ref):
    xv = x_ref[...]
    q_ref[...] = jnp.dot(xv, wl_ref[...], preferred_element_type=jnp.float32)
    r_ref[...] = jnp.dot(xv, wr_ref[...], preferred_element_type=jnp.float32)

  return pl.pallas_call(
      body,
      out_shape=(jax.ShapeDtypeStruct((n, dout), jnp.float32),
                 jax.ShapeDtypeStruct((n, dout), jnp.float32)),
  )(x, Wl, Wr)


def _tc_mid(part, deg2, r1, b1, Wl2, Wr2):
  n = r1.shape[0]
  dout = Wl2.shape[1]

  def body(p_ref, d_ref, r1_ref, b1_ref, wl_ref, wr_ref, q2_ref, r2_ref):
    deg = jnp.maximum(d_ref[0, :n] + d_ref[1, :n], 1.0)
    h = (p_ref[0, :n] + p_ref[1, :n]) / deg + b1_ref[...] + r1_ref[...]
    h = jnp.maximum(h, 0.0)
    q2_ref[...] = jnp.dot(h, wl_ref[...], preferred_element_type=jnp.float32)
    r2_ref[...] = jnp.dot(h, wr_ref[...], preferred_element_type=jnp.float32)

  return pl.pallas_call(
      body,
      out_shape=(jax.ShapeDtypeStruct((n, dout), jnp.float32),
                 jax.ShapeDtypeStruct((n, dout), jnp.float32)),
  )(part, deg2, r1, b1.reshape(1, -1), Wl2, Wr2)


def _tc_final(part, deg2, r2, b2):
  n, dout = r2.shape

  def body(p_ref, d_ref, r2_ref, b2_ref, o_ref):
    deg = jnp.maximum(d_ref[0, :n] + d_ref[1, :n], 1.0)
    o = (p_ref[0, :n] + p_ref[1, :n]) / deg + b2_ref[...] + r2_ref[...]
    m = jnp.max(o, axis=-1, keepdims=True)
    e = jnp.exp(o - m)
    lse = jnp.log(jnp.sum(e, axis=-1, keepdims=True)) + m
    o_ref[...] = o - lse

  return pl.pallas_call(
      body,
      out_shape=jax.ShapeDtypeStruct((n, dout), jnp.float32),
  )(part, deg2, r2, b2.reshape(1, -1))


def _edge_chunks(edge_index, ch, npad):
  e = edge_index.shape[1]
  nchunk = -(-e // (NW * ch))
  e_pad = NW * ch * nchunk
  src_r = jnp.pad(edge_index[0], (0, e_pad - e)).reshape(NW, nchunk, ch)
  dst_r = jnp.pad(edge_index[1], (0, e_pad - e),
                  constant_values=npad - 1).reshape(NW, nchunk, ch)
  return src_r, dst_r, nchunk


def kernel(x, edge_index, Wl1, Wr1, b1, Wl2, Wr2, b2):
  n, d_in = x.shape
  npad = ((n + NW * 16 - 1) // (NW * 16)) * (NW * 16)
  if npad == n:
    npad += NW * 16

  src_r, dst_r, nchunk = _edge_chunks(edge_index, CH, npad)
  ones_h = jnp.ones((CH,), jnp.float32)

  q1, r1 = _tc_transform(x, Wl1, Wr1)
  part1, degp = _sc_aggregate(q1, src_r, dst_r, ones_h,
                              npad, d_in, nchunk, CH, 1, with_deg=True)
  deg2 = degp.reshape(NC, npad, 1)
  q2, r2 = _tc_mid(part1, deg2, r1, b1, Wl2, Wr2)

  (part2,) = _sc_aggregate(q2, src_r, dst_r, ones_h,
                           npad, Wl2.shape[1], nchunk, CH, 2,
                           with_deg=False)
  return _tc_final(part2, deg2, r2, b2)


# speedup vs baseline: 1.6895x; 1.0286x over previous
import functools

import jax
import jax.numpy as jnp
from jax import lax
from jax.experimental import pallas as pl
from jax.experimental.pallas import tpu as pltpu
from jax.experimental.pallas import tpu_sc as plsc

NC = 2
NS = 16
NW = NC * NS
CH = 125


def _sc_aggregate(q, src_r, dst_r, ones_h, npad, d, nchunk, ch, slots,
                  with_deg):
  mesh = plsc.VectorSubcoreMesh(core_axis_name="c", subcore_axis_name="s",
                                num_cores=NC, num_subcores=NS)
  rows_per_tile = npad // NS

  out_type = [jax.ShapeDtypeStruct((NC, npad, d), jnp.float32)]
  scratch = [
      pltpu.VMEM((nchunk, ch), jnp.int32),
      pltpu.VMEM((nchunk, ch), jnp.int32),
      pltpu.VMEM((slots, ch, d), jnp.float32),
      pltpu.VMEM((16, d), jnp.float32),
      pltpu.VMEM_SHARED((npad, d), jnp.float32),
      pltpu.SemaphoreType.DMA((max(slots, 2),)),
      pltpu.SemaphoreType.DMA((max(slots, 2),)),
  ]
  if with_deg:
    out_type.append(jax.ShapeDtypeStruct((NC, npad), jnp.float32))
    scratch += [
        pltpu.VMEM((ch,), jnp.float32),
        pltpu.VMEM((rows_per_tile,), jnp.float32),
        pltpu.VMEM_SHARED((npad,), jnp.float32),
        pltpu.SemaphoreType.DMA,
    ]

  @functools.partial(
      pl.kernel, mesh=mesh, out_type=tuple(out_type),
      scratch_types=tuple(scratch),
      compiler_params=pltpu.CompilerParams(use_tc_tiling_on_sc=False),
      name="agg_deg" if with_deg else "agg")
  def agg_kernel(q_hbm, src_hbm, dst_hbm, ones_hbm, *rest):
    if with_deg:
      (part_hbm, degp_hbm, src_v, dst_v, rows_v, zmat_v, acc_s, gsems,
       ssems, ones_v, zrow_v, dega_s, dsem) = rest
    else:
      (part_hbm, src_v, dst_v, rows_v, zmat_v, acc_s, gsems, ssems) = rest
    c = lax.axis_index("c")
    s = lax.axis_index("s")
    wid = s * NC + c

    pltpu.sync_copy(src_hbm.at[wid], src_v)
    pltpu.sync_copy(dst_hbm.at[wid], dst_v)

    z16 = jnp.zeros((16,), jnp.float32)
    for i in range(16):
      for j in range(d // 16):
        zmat_v[i, pl.ds(j * 16, 16)] = z16
    base = s * rows_per_tile

    def clear_body(k, _):
      pltpu.sync_copy(zmat_v, acc_s.at[pl.ds(base + k * 16, 16)])
      return 0
    lax.fori_loop(0, rows_per_tile // 16, clear_body, 0)

    if with_deg:
      pltpu.sync_copy(ones_hbm, ones_v)
      for j in range(rows_per_tile // 16):
        zrow_v[pl.ds(j * 16, 16)] = z16
      pltpu.sync_copy(zrow_v, dega_s.at[pl.ds(base, rows_per_tile)])

    plsc.subcore_barrier()

    if slots == 3:
      def gw(j, k3):
        pltpu.make_async_copy(q_hbm.at[src_v.at[j]], rows_v.at[k3],
                              gsems.at[k3]).wait()

      def sw(j, k3):
        pltpu.make_async_copy(rows_v.at[k3], acc_s.at[dst_v.at[j]],
                              ssems.at[k3]).wait()

      def prime3(j, _):
        pltpu.async_copy(q_hbm.at[src_v.at[j]], rows_v.at[j], gsems.at[j])
        return 0
      lax.fori_loop(0, 2, prime3, 0)

      def chunk_body(j, _):
        k3 = lax.rem(j, 3)
        gw(j, k3)
        pltpu.async_copy(rows_v.at[k3], acc_s.at[dst_v.at[j]],
                         ssems.at[k3], add=True)

        @pl.when(j >= 1)
        def _():
          sw(j - 1, lax.rem(j - 1, 3))

        @pl.when(j + 2 < nchunk)
        def _():
          k3n = lax.rem(j + 2, 3)
          pltpu.async_copy(q_hbm.at[src_v.at[j + 2]], rows_v.at[k3n],
                           gsems.at[k3n])
        return 0
      lax.fori_loop(0, nchunk, chunk_body, 0)
      sw(nchunk - 1, lax.rem(nchunk - 1, 3))
      chunk_body = None
    elif slots == 2:
      pltpu.async_copy(q_hbm.at[src_v.at[0]], rows_v.at[0], gsems.at[0])

      def chunk_body(j, _):
        k = lax.rem(j, 2)

        @pl.when(j + 1 < nchunk)
        def _():
          pltpu.async_copy(q_hbm.at[src_v.at[j + 1]], rows_v.at[1 - k],
                           gsems.at[1 - k])

        pltpu.make_async_copy(q_hbm.at[src_v.at[j]], rows_v.at[k],
                              gsems.at[k]).wait()
        pltpu.sync_copy(rows_v.at[k], acc_s.at[dst_v.at[j]], add=True)
        if with_deg:
          pltpu.async_copy(ones_v, dega_s.at[dst_v.at[j]], dsem, add=True)
        return 0
    else:
      def chunk_body(j, _):
        pltpu.async_copy(q_hbm.at[src_v.at[j]], rows_v.at[0],
                         gsems.at[0]).wait()
        pltpu.sync_copy(rows_v.at[0], acc_s.at[dst_v.at[j]], add=True)
        if with_deg:
          pltpu.sync_copy(ones_v, dega_s.at[dst_v.at[j]], add=True)
        return 0
    if chunk_body is not None:
      lax.fori_loop(0, nchunk, chunk_body, 0)

    if with_deg and slots == 2:
      pltpu.make_async_copy(src_hbm.at[wid], dst_v, dsem).wait()

    plsc.subcore_barrier()

    pltpu.sync_copy(acc_s.at[pl.ds(base, rows_per_tile)],
                    part_hbm.at[c, pl.ds(base, rows_per_tile)])
    if with_deg:
      pltpu.sync_copy(dega_s.at[pl.ds(base, rows_per_tile)],
                      degp_hbm.at[c, pl.ds(base, rows_per_tile)])

  return agg_kernel(q, src_r, dst_r, ones_h)


def _tc_transform(x, Wl, Wr):
  n, _ = x.shape
  dout = Wl.shape[1]

  def body(x_ref, wl_ref, wr_ref, q_ref, r_ref):
    xv = x_ref[...]
    q_ref[...] = jnp.dot(xv, wl_ref[...], preferred_element_type=jnp.float32)
    r_ref[...] = jnp.dot(xv, wr_ref[...], preferred_element_type=jnp.float32)

  return pl.pallas_call(
      body,
      out_shape=(jax.ShapeDtypeStruct((n, dout), jnp.float32),
                 jax.ShapeDtypeStruct((n, dout), jnp.float32)),
  )(x, Wl, Wr)


def _tc_mid(part, deg2, r1, b1, Wl2, Wr2):
  n = r1.shape[0]
  dout = Wl2.shape[1]

  def body(p_ref, d_ref, r1_ref, b1_ref, wl_ref, wr_ref, q2_ref, r2_ref):
    deg = jnp.maximum(d_ref[0, :n] + d_ref[1, :n], 1.0)
    h = (p_ref[0, :n] + p_ref[1, :n]) / deg + b1_ref[...] + r1_ref[...]
    h = jnp.maximum(h, 0.0)
    q2_ref[...] = jnp.dot(h, wl_ref[...], preferred_element_type=jnp.float32)
    r2_ref[...] = jnp.dot(h, wr_ref[...], preferred_element_type=jnp.float32)

  return pl.pallas_call(
      body,
      out_shape=(jax.ShapeDtypeStruct((n, dout), jnp.float32),
                 jax.ShapeDtypeStruct((n, dout), jnp.float32)),
  )(part, deg2, r1, b1.reshape(1, -1), Wl2, Wr2)


def _tc_final(part, deg2, r2, b2):
  n, dout = r2.shape

  def body(p_ref, d_ref, r2_ref, b2_ref, o_ref):
    deg = jnp.maximum(d_ref[0, :n] + d_ref[1, :n], 1.0)
    o = (p_ref[0, :n] + p_ref[1, :n]) / deg + b2_ref[...] + r2_ref[...]
    m = jnp.max(o, axis=-1, keepdims=True)
    e = jnp.exp(o - m)
    lse = jnp.log(jnp.sum(e, axis=-1, keepdims=True)) + m
    o_ref[...] = o - lse

  return pl.pallas_call(
      body,
      out_shape=jax.ShapeDtypeStruct((n, dout), jnp.float32),
  )(part, deg2, r2, b2.reshape(1, -1))


def _edge_chunks(edge_index, ch, npad):
  e = edge_index.shape[1]
  nchunk = -(-e // (NW * ch))
  e_pad = NW * ch * nchunk
  src_r = jnp.pad(edge_index[0], (0, e_pad - e)).reshape(NW, nchunk, ch)
  dst_r = jnp.pad(edge_index[1], (0, e_pad - e),
                  constant_values=npad - 1).reshape(NW, nchunk, ch)
  return src_r, dst_r, nchunk


def kernel(x, edge_index, Wl1, Wr1, b1, Wl2, Wr2, b2):
  n, d_in = x.shape
  npad = ((n + NW * 16 - 1) // (NW * 16)) * (NW * 16)
  if npad == n:
    npad += NW * 16

  src_r, dst_r, nchunk = _edge_chunks(edge_index, CH, npad)
  ones_h = jnp.ones((CH,), jnp.float32)

  q1, r1 = _tc_transform(x, Wl1, Wr1)
  part1, degp = _sc_aggregate(q1, src_r, dst_r, ones_h,
                              npad, d_in, nchunk, CH, 1, with_deg=True)
  deg2 = degp.reshape(NC, npad, 1)
  q2, r2 = _tc_mid(part1, deg2, r1, b1, Wl2, Wr2)

  (part2,) = _sc_aggregate(q2, src_r, dst_r, ones_h,
                           npad, Wl2.shape[1], nchunk, CH, 3,
                           with_deg=False)
  return _tc_final(part2, deg2, r2, b2)
